# Initial kernel scaffold; baseline (speedup 1.0000x reference)
import jax
import jax.numpy as jnp
from jax.experimental import pallas as pl


def kernel(features, coors, coors_inv, scale_2_coors_inv, W1, b1, Wp1, bp1, g1, beta1, Wp2, bp2, g2, beta2, Wp3, bp3, Wo1, bo1, Wo2, bo2):
    raise NotImplementedError("write your pallas kernel here")


# speedup vs baseline: 1.1947x; 1.1947x over previous
import functools

import jax
import jax.numpy as jnp
from jax import lax
from jax.experimental import pallas as pl
from jax.experimental.pallas import tpu as pltpu

_BLK = 2000


def _leaky(x):
    return jnp.where(x >= 0, x, 0.1 * x)


def _p1_body(m_ref, f_ref, ds_ref, w1_ref, b1_ref, wo1t_ref, wp1_ref, bp1_ref,
             a1_ref, x1_ref, st_ref):
    step = pl.program_id(0)
    ident = _leaky(jnp.dot(f_ref[...], w1_ref[...],
                           preferred_element_type=jnp.float32) + b1_ref[0:1, :])
    a1_ref[...] = jnp.dot(ident, wo1t_ref[...], preferred_element_type=jnp.float32)
    x1 = _leaky(jnp.dot(ds_ref[...], wp1_ref[...],
                        preferred_element_type=jnp.float32) + bp1_ref[0:1, :])
    x1_ref[...] = x1
    row = step * _BLK + lax.broadcasted_iota(jnp.int32, (_BLK, 1), 0)
    mask = (row < m_ref[0]).astype(jnp.float32)
    x1m = x1 * mask

    @pl.when(step == 0)
    def _():
        st_ref[...] = jnp.zeros_like(st_ref)

    st_ref[0:1, :] += jnp.sum(x1m, axis=0, keepdims=True)
    st_ref[1:2, :] += jnp.sum(x1m * x1, axis=0, keepdims=True)


def _p2_body(m_ref, x1_ref, a_ref, b_ref, wp2_ref, bp2_ref, x2_ref, st_ref):
    step = pl.program_id(0)
    y1 = x1_ref[...] * a_ref[0:1, :] + b_ref[0:1, :]
    x2 = _leaky(jnp.dot(y1, wp2_ref[...],
                        preferred_element_type=jnp.float32) + bp2_ref[0:1, :])
    x2_ref[...] = x2
    row = step * _BLK + lax.broadcasted_iota(jnp.int32, (_BLK, 1), 0)
    mask = (row < m_ref[0]).astype(jnp.float32)
    x2m = x2 * mask

    @pl.when(step == 0)
    def _():
        st_ref[...] = jnp.zeros_like(st_ref)

    st_ref[0:1, :] += jnp.sum(x2m, axis=0, keepdims=True)
    st_ref[1:2, :] += jnp.sum(x2m * x2, axis=0, keepdims=True)


def _p3_body(x2_ref, a_ref, b_ref, wp3_ref, bp3_ref, h_ref):
    y2 = x2_ref[...] * a_ref[0:1, :] + b_ref[0:1, :]
    h_ref[...] = _leaky(jnp.dot(y2, wp3_ref[...],
                                preferred_element_type=jnp.float32) + bp3_ref[0:1, :])


def _p4_body(a1_ref, hg_ref, wo1b_ref, bo1_ref, wo2_ref, bo2_ref, z_ref):
    u = a1_ref[...] + jnp.dot(hg_ref[...], wo1b_ref[...],
                              preferred_element_type=jnp.float32)
    z1 = _leaky(u + bo1_ref[0:1, :])
    z_ref[...] = jnp.dot(z1, wo2_ref[...],
                         preferred_element_type=jnp.float32) + bo2_ref[0:1, :]


def _row8(v):
    return jnp.broadcast_to(v[None, :], (8, v.shape[0]))


def _full(shape):
    return pl.BlockSpec(shape, lambda i: (0, 0))


def _rows(c):
    return pl.BlockSpec((_BLK, c), lambda i: (i, 0))


def kernel(features, coors, coors_inv, scale_2_coors_inv,
           W1, b1, Wp1, bp1, g1, beta1, Wp2, bp2, g2, beta2, Wp3, bp3,
           Wo1, bo1, Wo2, bo2):
    n = features.shape[0]
    in_c = features.shape[1]
    out_c = W1.shape[1]
    hid = Wp1.shape[1]
    m_out = 25000
    grid = n // _BLK

    c32 = coors.astype(jnp.int32)
    key = ((c32[:, 0] << 27) | ((c32[:, 1] >> 1) << 18)
           | ((c32[:, 2] >> 1) << 9) | (c32[:, 3] >> 1))
    sk, perm = lax.sort_key_val(key, jnp.arange(n, dtype=jnp.int32))
    newseg = jnp.concatenate(
        [jnp.ones((1,), jnp.int32), (sk[1:] != sk[:-1]).astype(jnp.int32)])
    ranks = jnp.cumsum(newseg) - 1
    inv = jnp.zeros((n,), jnp.int32).at[perm].set(ranks)
    n_valid = (ranks[-1] + 1).astype(jnp.float32)
    m_arr = ranks[-1:] + 1

    cnt = jax.ops.segment_sum(jnp.ones((n,), jnp.float32), inv, num_segments=n)
    sums = jax.ops.segment_sum(features, inv, num_segments=n)
    ds = sums / jnp.maximum(cnt, 1.0)[:, None]

    wo1t = Wo1[:out_c]
    wo1b = Wo1[out_c:]

    a1, x1, st1 = pl.pallas_call(
        _p1_body,
        grid=(grid,),
        in_specs=[
            pl.BlockSpec(memory_space=pltpu.SMEM),
            _rows(in_c), _rows(in_c), _full((in_c, out_c)), _full((8, out_c)),
            _full((out_c, out_c)), _full((in_c, hid)), _full((8, hid)),
        ],
        out_specs=[_rows(out_c), _rows(hid),
                   pl.BlockSpec((8, hid), lambda i: (0, 0))],
        out_shape=[
            jax.ShapeDtypeStruct((n, out_c), jnp.float32),
            jax.ShapeDtypeStruct((n, hid), jnp.float32),
            jax.ShapeDtypeStruct((8, hid), jnp.float32),
        ],
        compiler_params=pltpu.CompilerParams(
            dimension_semantics=("arbitrary",)),
    )(m_arr, features, ds, W1, _row8(b1), wo1t, Wp1, _row8(bp1))

    mean1 = st1[0] / n_valid
    var1 = jnp.maximum(st1[1] / n_valid - mean1 * mean1, 0.0)
    sc1 = g1 / jnp.sqrt(var1 + 1e-5)
    sh1 = beta1 - mean1 * sc1

    x2, st2 = pl.pallas_call(
        _p2_body,
        grid=(grid,),
        in_specs=[
            pl.BlockSpec(memory_space=pltpu.SMEM),
            _rows(hid), _full((8, hid)), _full((8, hid)),
            _full((hid, hid)), _full((8, hid)),
        ],
        out_specs=[_rows(hid), pl.BlockSpec((8, hid), lambda i: (0, 0))],
        out_shape=[
            jax.ShapeDtypeStruct((n, hid), jnp.float32),
            jax.ShapeDtypeStruct((8, hid), jnp.float32),
        ],
        compiler_params=pltpu.CompilerParams(
            dimension_semantics=("arbitrary",)),
    )(m_arr, x1, _row8(sc1), _row8(sh1), Wp2, _row8(bp2))

    mean2 = st2[0] / n_valid
    var2 = jnp.maximum(st2[1] / n_valid - mean2 * mean2, 0.0)
    sc2 = g2 / jnp.sqrt(var2 + 1e-5)
    sh2 = beta2 - mean2 * sc2

    h = pl.pallas_call(
        _p3_body,
        grid=(grid,),
        in_specs=[_rows(hid), _full((8, hid)), _full((8, hid)),
                  _full((hid, out_c)), _full((8, out_c))],
        out_specs=_rows(out_c),
        out_shape=jax.ShapeDtypeStruct((n, out_c), jnp.float32),
        compiler_params=pltpu.CompilerParams(
            dimension_semantics=("arbitrary",)),
    )(x2, _row8(sc2), _row8(sh2), Wp3, _row8(bp3))

    hg = h[inv]

    z = pl.pallas_call(
        _p4_body,
        grid=(grid,),
        in_specs=[_rows(out_c), _rows(out_c), _full((out_c, out_c)),
                  _full((8, out_c)), _full((out_c, out_c)), _full((8, out_c))],
        out_specs=_rows(out_c),
        out_shape=jax.ShapeDtypeStruct((n, out_c), jnp.float32),
        compiler_params=pltpu.CompilerParams(
            dimension_semantics=("arbitrary",)),
    )(a1, hg, wo1b, _row8(bo1), Wo2, _row8(bo2))

    zs = jax.ops.segment_sum(z[coors_inv], scale_2_coors_inv, num_segments=m_out)
    cnt2 = jax.ops.segment_sum(jnp.ones((coors_inv.shape[0],), jnp.float32),
                               scale_2_coors_inv, num_segments=m_out)
    return zs / jnp.maximum(cnt2, 1.0)[:, None]
